# baseline (speedup 1.0000x reference)
import functools

import jax
import jax.numpy as jnp
from jax import lax
from jax.experimental import pallas as pl
from jax.experimental.pallas import tpu as pltpu
from jax.experimental.pallas import tpu_sc as plsc

N_EMBD = 1024
ROWS = 4 * 8192
NW = 32
ROWS_PER_W = ROWS // NW
CHUNK = 8
NCHUNK = ROWS_PER_W // CHUNK
NB = 14
G = 6

_mesh = plsc.VectorSubcoreMesh(core_axis_name="c", subcore_axis_name="s")


@functools.partial(
    pl.kernel,
    out_type=jax.ShapeDtypeStruct((ROWS, N_EMBD), jnp.float32),
    mesh=_mesh,
    scratch_types=[
        pltpu.VMEM((ROWS_PER_W,), jnp.int32),
        pltpu.VMEM((NB, CHUNK, N_EMBD), jnp.float32),
        pltpu.SemaphoreType.DMA,
        pltpu.SemaphoreType.DMA,
        pltpu.SemaphoreType.DMA,
        pltpu.SemaphoreType.DMA,
        pltpu.SemaphoreType.DMA,
        pltpu.SemaphoreType.DMA,
        pltpu.SemaphoreType.DMA,
        pltpu.SemaphoreType.DMA,
        pltpu.SemaphoreType.DMA,
        pltpu.SemaphoreType.DMA,
        pltpu.SemaphoreType.DMA,
        pltpu.SemaphoreType.DMA,
        pltpu.SemaphoreType.DMA,
        pltpu.SemaphoreType.DMA,
        pltpu.SemaphoreType.DMA,
        pltpu.SemaphoreType.DMA,
        pltpu.SemaphoreType.DMA,
        pltpu.SemaphoreType.DMA,
        pltpu.SemaphoreType.DMA,
        pltpu.SemaphoreType.DMA,
        pltpu.SemaphoreType.DMA,
        pltpu.SemaphoreType.DMA,
        pltpu.SemaphoreType.DMA,
        pltpu.SemaphoreType.DMA,
        pltpu.SemaphoreType.DMA,
        pltpu.SemaphoreType.DMA,
        pltpu.SemaphoreType.DMA,
        pltpu.SemaphoreType.DMA,
    ],
)
def _embed_sc(ids_hbm, table_hbm, out_hbm, idx_v, bufs, *sems):
    gsems = sems[:NB]
    wsems = sems[NB:]
    wid = lax.axis_index("s") * 2 + lax.axis_index("c")
    base = wid * ROWS_PER_W
    pltpu.sync_copy(ids_hbm.at[pl.ds(base, ROWS_PER_W)], idx_v)

    gd = [None] * NCHUNK
    wd = [None] * NCHUNK
    for i in range(NCHUNK):
        b = i % NB
        if i >= NB:
            wd[i - NB].wait()
        gd[i] = pltpu.async_copy(
            table_hbm.at[idx_v.at[pl.ds(i * CHUNK, CHUNK)]], bufs.at[b],
            gsems[b],
        )
        if i >= G:
            j = i - G
            gd[j].wait()
            wd[j] = pltpu.async_copy(
                bufs.at[j % NB], out_hbm.at[pl.ds(base + j * CHUNK, CHUNK)],
                wsems[j % NB],
            )
    for j in range(NCHUNK - G, NCHUNK):
        gd[j].wait()
        wd[j] = pltpu.async_copy(
            bufs.at[j % NB], out_hbm.at[pl.ds(base + j * CHUNK, CHUNK)],
            wsems[j % NB],
        )
    for j in range(NCHUNK - NB, NCHUNK):
        wd[j].wait()


def kernel(input_ids, wte):
    ids = input_ids.reshape(-1).astype(jnp.int32)
    flat = _embed_sc(ids, wte)
    return flat.reshape(input_ids.shape[0], input_ids.shape[1], N_EMBD)
